# baseline (speedup 1.0000x reference)
import jax
import jax.numpy as jnp
from jax.experimental import pallas as pl
from jax.experimental.pallas import tpu as pltpu

_B, _N, _D = 32, 2048, 512
_D_LAT = 128
_CHUNK = 256
_NCHUNK = _N // _CHUNK
_HB = _B // 2

_EPS_CACHE = []


def _eps_const():
    if not _EPS_CACHE:
        try:
            with jax.ensure_compile_time_eval():
                eps = jax.random.normal(
                    jax.random.key(42), (_B, _D_LAT), dtype=jnp.float32)
        except Exception:
            eps = jax.random.normal(
                jax.random.key(42), (_B, _D_LAT), dtype=jnp.float32)
        _EPS_CACHE.append(eps)
    return _EPS_CACHE[0]


def _pool_mlp_kernel(h0_ref, h1_ref, wagg_ref, bagg_ref, wbot_ref, bbot_ref,
                     wmu_ref, bmu_ref, wlv_ref, blv_ref, eps_ref,
                     z_ref, mu_ref, lv_ref, sum_ref, max_ref):
    i = pl.program_id(0)
    p0s = jnp.sum(h0_ref[...], axis=1)
    p0m = jnp.max(h0_ref[...], axis=1)
    p1s = jnp.sum(h1_ref[...], axis=1)
    p1m = jnp.max(h1_ref[...], axis=1)

    @pl.when(i == 0)
    def _():
        sum_ref[0:_HB] = p0s
        max_ref[0:_HB] = p0m
        sum_ref[_HB:_B] = p1s
        max_ref[_HB:_B] = p1m

    @pl.when(i > 0)
    def _():
        sum_ref[0:_HB] += p0s
        max_ref[0:_HB] = jnp.maximum(max_ref[0:_HB], p0m)
        sum_ref[_HB:_B] += p1s
        max_ref[_HB:_B] = jnp.maximum(max_ref[_HB:_B], p1m)

    @pl.when(i == _NCHUNK - 1)
    def _():
        mean = sum_ref[...] * (1.0 / _N)
        mx = max_ref[...]
        g = (jnp.dot(mean, wagg_ref[0:_D, :], preferred_element_type=jnp.float32)
             + jnp.dot(mx, wagg_ref[_D:2 * _D, :], preferred_element_type=jnp.float32)
             + bagg_ref[...])
        bvec = jnp.maximum(
            jnp.dot(g, wbot_ref[...], preferred_element_type=jnp.float32) + bbot_ref[...], 0.0)
        mu = jnp.dot(bvec, wmu_ref[...], preferred_element_type=jnp.float32) + bmu_ref[...]
        lv = jnp.dot(bvec, wlv_ref[...], preferred_element_type=jnp.float32) + blv_ref[...]
        mu_ref[...] = mu
        lv_ref[...] = lv
        z_ref[...] = mu + eps_ref[...] * jnp.exp(0.5 * lv)


def kernel(h, W_agg, b_agg, W_bot, b_bot, W_mu, b_mu, W_lv, b_lv):
    full = lambda shape: pl.BlockSpec(shape, lambda i: (0,) * len(shape))
    z, mu, lv = pl.pallas_call(
        _pool_mlp_kernel,
        grid=(_NCHUNK,),
        in_specs=[
            pl.BlockSpec((_HB, _CHUNK, _D), lambda i: (0, i, 0)),
            pl.BlockSpec((_HB, _CHUNK, _D), lambda i: (1, i, 0)),
            full((2 * _D, _D)),
            full((1, _D)),
            full((_D, 256)),
            full((1, 256)),
            full((256, _D_LAT)),
            full((1, _D_LAT)),
            full((256, _D_LAT)),
            full((1, _D_LAT)),
            full((_B, _D_LAT)),
        ],
        out_specs=[full((_B, _D_LAT))] * 3,
        out_shape=[jax.ShapeDtypeStruct((_B, _D_LAT), jnp.float32)] * 3,
        scratch_shapes=[pltpu.VMEM((_B, _D), jnp.float32),
                        pltpu.VMEM((_B, _D), jnp.float32)],
        compiler_params=pltpu.CompilerParams(
            dimension_semantics=("arbitrary",)),
    )(h, h, W_agg, b_agg.reshape(1, -1), W_bot, b_bot.reshape(1, -1),
      W_mu, b_mu.reshape(1, -1), W_lv, b_lv.reshape(1, -1), _eps_const())
    return (z, mu, lv)


# speedup vs baseline: 1.0253x; 1.0253x over previous
import jax
import jax.numpy as jnp
from jax.experimental import pallas as pl
from jax.experimental.pallas import tpu as pltpu

_B, _N, _D = 32, 2048, 512
_D_LAT = 128
_CHUNK = 128
_NCHUNK = _N // _CHUNK
_HB = _B // 2

_EPS_CACHE = []


def _eps_const():
    if not _EPS_CACHE:
        try:
            with jax.ensure_compile_time_eval():
                eps = jax.random.normal(
                    jax.random.key(42), (_B, _D_LAT), dtype=jnp.float32)
        except Exception:
            eps = jax.random.normal(
                jax.random.key(42), (_B, _D_LAT), dtype=jnp.float32)
        _EPS_CACHE.append(eps)
    return _EPS_CACHE[0]


def _pool_mlp_kernel(h0_ref, h1_ref, wagg_ref, bagg_ref, wbot_ref, bbot_ref,
                     wmu_ref, bmu_ref, wlv_ref, blv_ref, eps_ref,
                     z_ref, mu_ref, lv_ref, sum_ref, max_ref):
    i = pl.program_id(0)
    p0s = jnp.sum(h0_ref[...], axis=1)
    p0m = jnp.max(h0_ref[...], axis=1)
    p1s = jnp.sum(h1_ref[...], axis=1)
    p1m = jnp.max(h1_ref[...], axis=1)

    @pl.when(i == 0)
    def _():
        sum_ref[0:_HB] = p0s
        max_ref[0:_HB] = p0m
        sum_ref[_HB:_B] = p1s
        max_ref[_HB:_B] = p1m

    @pl.when(i > 0)
    def _():
        sum_ref[0:_HB] += p0s
        max_ref[0:_HB] = jnp.maximum(max_ref[0:_HB], p0m)
        sum_ref[_HB:_B] += p1s
        max_ref[_HB:_B] = jnp.maximum(max_ref[_HB:_B], p1m)

    @pl.when(i == _NCHUNK - 1)
    def _():
        mean = sum_ref[...] * (1.0 / _N)
        mx = max_ref[...]
        g = (jnp.dot(mean, wagg_ref[0:_D, :], preferred_element_type=jnp.float32)
             + jnp.dot(mx, wagg_ref[_D:2 * _D, :], preferred_element_type=jnp.float32)
             + bagg_ref[...])
        bvec = jnp.maximum(
            jnp.dot(g, wbot_ref[...], preferred_element_type=jnp.float32) + bbot_ref[...], 0.0)
        mu = jnp.dot(bvec, wmu_ref[...], preferred_element_type=jnp.float32) + bmu_ref[...]
        lv = jnp.dot(bvec, wlv_ref[...], preferred_element_type=jnp.float32) + blv_ref[...]
        mu_ref[...] = mu
        lv_ref[...] = lv
        z_ref[...] = mu + eps_ref[...] * jnp.exp(0.5 * lv)


def kernel(h, W_agg, b_agg, W_bot, b_bot, W_mu, b_mu, W_lv, b_lv):
    full = lambda shape: pl.BlockSpec(shape, lambda i: (0,) * len(shape))
    z, mu, lv = pl.pallas_call(
        _pool_mlp_kernel,
        grid=(_NCHUNK,),
        in_specs=[
            pl.BlockSpec((_HB, _CHUNK, _D), lambda i: (0, i, 0)),
            pl.BlockSpec((_HB, _CHUNK, _D), lambda i: (1, i, 0)),
            full((2 * _D, _D)),
            full((1, _D)),
            full((_D, 256)),
            full((1, 256)),
            full((256, _D_LAT)),
            full((1, _D_LAT)),
            full((256, _D_LAT)),
            full((1, _D_LAT)),
            full((_B, _D_LAT)),
        ],
        out_specs=[full((_B, _D_LAT))] * 3,
        out_shape=[jax.ShapeDtypeStruct((_B, _D_LAT), jnp.float32)] * 3,
        scratch_shapes=[pltpu.VMEM((_B, _D), jnp.float32),
                        pltpu.VMEM((_B, _D), jnp.float32)],
        compiler_params=pltpu.CompilerParams(
            dimension_semantics=("arbitrary",)),
    )(h, h, W_agg, b_agg.reshape(1, -1), W_bot, b_bot.reshape(1, -1),
      W_mu, b_mu.reshape(1, -1), W_lv, b_lv.reshape(1, -1), _eps_const())
    return (z, mu, lv)
